# baseline (speedup 1.0000x reference)
import functools

import jax
import jax.numpy as jnp
from jax import lax
from jax.experimental import pallas as pl
from jax.experimental.pallas import tpu as pltpu
from jax.experimental.pallas import tpu_sc as plsc

_TOKENS = 16384
_D = 4096
_E = 64
_K = 8
_BT = 1024

_NC = 2
_NS = 16
_NW = _NC * _NS
_CHUNK = _TOKENS // _NW
_GROUPS = _CHUNK // 16


def _logits_t_kernel(x_ref, w_ref, b_ref, out_ref):
    out_ref[:] = (
        lax.dot_general(
            w_ref[:],
            x_ref[:],
            (((1,), (1,)), ((), ())),
            preferred_element_type=jnp.float32,
        )
        + b_ref[:]
    )


def _tc_logits_t(x, W, b):
    b2 = b.reshape(_E, 1)
    return pl.pallas_call(
        _logits_t_kernel,
        grid=(_TOKENS // _BT,),
        in_specs=[
            pl.BlockSpec((_BT, _D), lambda i: (i, 0)),
            pl.BlockSpec((_E, _D), lambda i: (0, 0)),
            pl.BlockSpec((_E, 1), lambda i: (0, 0)),
        ],
        out_specs=pl.BlockSpec((_E, _BT), lambda i: (0, i)),
        out_shape=jax.ShapeDtypeStruct((_E, _TOKENS), jnp.float32),
        compiler_params=pltpu.CompilerParams(
            dimension_semantics=("parallel",),
        ),
    )(x, W, b2)


def _sc_gate_body(lt_hbm, out_hbm, chunk_ref, outc_ref):
    wid = lax.axis_index("s") * _NC + lax.axis_index("c")
    base = wid * _CHUNK
    pltpu.sync_copy(lt_hbm.at[:, pl.ds(base, _CHUNK)], chunk_ref)

    neg_inf = jnp.full((16,), -jnp.inf, jnp.float32)

    def group(g, carry):
        sl = pl.ds(g * 16, 16)
        top = [neg_inf] * _K
        for e in range(_E):
            v = chunk_ref[e, sl]
            for k in range(_K):
                hi = jnp.maximum(top[k], v)
                v = jnp.minimum(top[k], v)
                top[k] = hi
        t0 = top[0]
        t7 = top[_K - 1]
        z = jnp.zeros((16,), jnp.float32)
        s = jnp.zeros((16,), jnp.float32)
        for e in range(_E):
            v = chunk_ref[e, sl]
            ex = jnp.exp(v - t0)
            outc_ref[e, sl] = ex
            z = z + ex
            s = s + jnp.where(v >= t7, ex, 0.0)
        rden = 1.0 / (s + 1e-6 * z)
        e8 = jnp.exp(t7 - t0)
        for e in range(_E):
            ex = outc_ref[e, sl]
            outc_ref[e, sl] = jnp.where(ex >= e8, ex * rden, 0.0)
        return carry

    lax.fori_loop(0, _GROUPS, group, 0)
    pltpu.sync_copy(outc_ref, out_hbm.at[:, pl.ds(base, _CHUNK)])


def _sc_gate(logits_t):
    mesh = plsc.VectorSubcoreMesh(core_axis_name="c", subcore_axis_name="s")
    f = functools.partial(
        pl.kernel,
        mesh=mesh,
        out_type=jax.ShapeDtypeStruct((_E, _TOKENS), jnp.float32),
        scratch_types=[
            pltpu.VMEM((_E, _CHUNK), jnp.float32),
            pltpu.VMEM((_E, _CHUNK), jnp.float32),
        ],
    )(_sc_gate_body)
    return f(logits_t)


def _transpose_kernel(in_ref, out_ref):
    out_ref[:] = in_ref[:].T


def _tc_transpose(out_t):
    return pl.pallas_call(
        _transpose_kernel,
        grid=(_TOKENS // _BT,),
        in_specs=[pl.BlockSpec((_E, _BT), lambda i: (0, i))],
        out_specs=pl.BlockSpec((_BT, _E), lambda i: (i, 0)),
        out_shape=jax.ShapeDtypeStruct((_TOKENS, _E), jnp.float32),
        compiler_params=pltpu.CompilerParams(
            dimension_semantics=("parallel",),
        ),
    )(out_t)


def kernel(x, W, b):
    return _tc_transpose(_sc_gate(_tc_logits_t(x, W, b)))


# speedup vs baseline: 1.3767x; 1.3767x over previous
import jax
import jax.numpy as jnp
from jax.experimental import pallas as pl
from jax.experimental.pallas import tpu as pltpu

_TOKENS = 16384
_D = 4096
_E = 64
_K = 8
_BT = 2048
_DH = _D // 2


def _gate_kernel(x_ref, w_ref, b_ref, out_ref, acc_ref):
    j = pl.program_id(1)
    part = jax.lax.dot_general(
        x_ref[:],
        w_ref[:],
        (((1,), (1,)), ((), ())),
        preferred_element_type=jnp.float32,
    )

    @pl.when(j == 0)
    def _():
        acc_ref[:] = part

    @pl.when(j == 1)
    def _():
        logits = acc_ref[:] + part + b_ref[:]
        cur = logits
        mask = jnp.zeros_like(logits)
        for _ in range(_K):
            m = jnp.max(cur, axis=1, keepdims=True)
            sel = cur >= m
            mask = jnp.where(sel, 1.0, mask)
            cur = jnp.where(sel, -jnp.inf, cur)
        row_max = jnp.max(logits, axis=1, keepdims=True)
        e = jnp.exp(logits - row_max)
        z = jnp.sum(e, axis=1, keepdims=True)
        s = jnp.sum(e * mask, axis=1, keepdims=True)
        out_ref[:] = (e * mask) / (s + 1e-6 * z)


def kernel(x, W, b):
    b2 = b.reshape(1, _E)
    return pl.pallas_call(
        _gate_kernel,
        grid=(_TOKENS // _BT, 2),
        in_specs=[
            pl.BlockSpec((_BT, _DH), lambda i, j: (i, j)),
            pl.BlockSpec((_E, _DH), lambda i, j: (0, j)),
            pl.BlockSpec((1, _E), lambda i, j: (0, 0)),
        ],
        out_specs=pl.BlockSpec((_BT, _E), lambda i, j: (i, 0)),
        out_shape=jax.ShapeDtypeStruct((_TOKENS, _E), jnp.float32),
        scratch_shapes=[pltpu.VMEM((_BT, _E), jnp.float32)],
        compiler_params=pltpu.CompilerParams(
            dimension_semantics=("parallel", "arbitrary"),
        ),
    )(x, W, b2)
